# Initial kernel scaffold; baseline (speedup 1.0000x reference)
import jax
import jax.numpy as jnp
from jax.experimental import pallas as pl


def kernel(x, graph_pool, padded_nei, adj, Ws0, bs0, g_mlp, be_mlp, Ws1, bs1, g_out, be_out):
    raise NotImplementedError("write your pallas kernel here")


# speedup vs baseline: 1.0059x; 1.0059x over previous
import functools

import jax
import jax.numpy as jnp
from jax.experimental import pallas as pl

_EPS = 1e-5
_WIN = 560


def _agg_kernel(adj_ref, h_ref, w0_ref, b_ref, out_ref):
    bf16 = jnp.bfloat16
    pooled = jnp.dot(adj_ref[...].astype(bf16), h_ref[...].astype(bf16),
                     preferred_element_type=jnp.float32)
    out_ref[...] = jax.lax.dot_general(
        pooled.astype(bf16), w0_ref[...].astype(bf16),
        (((1,), (1,)), ((), ())),
        preferred_element_type=jnp.float32) + b_ref[...]


def _colsum(t):
    acc = jnp.zeros((1, t.shape[1]), jnp.float32)
    for r0 in range(0, t.shape[0], _WIN):
        acc = acc + jnp.sum(t[r0:r0 + _WIN], axis=0, keepdims=True)
    return acc


def _bn_relu(t, g_ref, b_ref):
    recip = jnp.float32(1.0) / jnp.float32(t.shape[0])
    mu = _colsum(t) * recip
    d = t - mu
    var = _colsum(d * d) * recip
    return jnp.maximum(
        d / jnp.sqrt(var + _EPS) * g_ref[...] + b_ref[...], 0.0)


def _mlp_mid_kernel(t_ref, gm_ref, bm_ref, w1_ref, b1_ref, go_ref, bo_ref,
                    h_ref):
    u = _bn_relu(t_ref[...], gm_ref, bm_ref)
    v = jax.lax.dot_general(
        u, w1_ref[...], (((1,), (1,)), ((), ())),
        preferred_element_type=jnp.float32) + b1_ref[...]
    h_ref[...] = _bn_relu(v, go_ref, bo_ref)


def _mlp_final_kernel(t_ref, gm_ref, bm_ref, w1_ref, b1_ref, go_ref, bo_ref,
                      gp_ref, h_ref, ph_ref):
    u = _bn_relu(t_ref[...], gm_ref, bm_ref)
    v = jax.lax.dot_general(
        u, w1_ref[...], (((1,), (1,)), ((), ())),
        preferred_element_type=jnp.float32) + b1_ref[...]
    h = _bn_relu(v, go_ref, bo_ref)
    h_ref[...] = h
    ph_ref[...] = jnp.dot(gp_ref[...], h, preferred_element_type=jnp.float32)


def kernel(x, graph_pool, padded_nei, adj, Ws0, bs0, g_mlp, be_mlp,
           Ws1, bs1, g_out, be_out):
    del padded_nei
    N, D = x.shape
    G = graph_pool.shape[0]
    L = Ws0.shape[0]
    Bi = 400 if N % 400 == 0 else N

    f32 = jnp.float32
    row = lambda a: a.reshape(1, D)

    h = x
    ph = None
    for l in range(L):
        if l == 0:
            t = (adj @ h) @ Ws0[0].T + bs0[0]
        else:
            t = pl.pallas_call(
                _agg_kernel,
                grid=(N // Bi,),
                in_specs=[
                    pl.BlockSpec((Bi, N), lambda i: (i, 0)),
                    pl.BlockSpec((N, D), lambda i: (0, 0)),
                    pl.BlockSpec((D, D), lambda i: (0, 0)),
                    pl.BlockSpec((1, D), lambda i: (0, 0)),
                ],
                out_specs=pl.BlockSpec((Bi, D), lambda i: (i, 0)),
                out_shape=jax.ShapeDtypeStruct((N, D), f32),
            )(adj, h, Ws0[l], bs0[l].reshape(1, D))

        if l + 1 < L:
            h = pl.pallas_call(
                _mlp_mid_kernel,
                out_shape=jax.ShapeDtypeStruct((N, D), f32),
            )(t, row(g_mlp[l]), row(be_mlp[l]), Ws1[l], row(bs1[l]),
              row(g_out[l]), row(be_out[l]))
        else:
            h, ph = pl.pallas_call(
                _mlp_final_kernel,
                out_shape=(jax.ShapeDtypeStruct((N, D), f32),
                           jax.ShapeDtypeStruct((G, D), f32)),
            )(t, row(g_mlp[l]), row(be_mlp[l]), Ws1[l], row(bs1[l]),
              row(g_out[l]), row(be_out[l]), graph_pool)

    return (ph, h)
